# baseline (speedup 1.0000x reference)
import functools

import jax
import jax.numpy as jnp
from jax import lax
from jax.experimental import pallas as pl
from jax.experimental.pallas import tpu as pltpu
from jax.experimental.pallas import tpu_sc as plsc

_B = 16
_D = 256
_HW = 1024
_K = 2048
_KC = 256
_NKC = _K // _KC
_BETA = 0.25

_NC = 2
_NS = 16
_NW = _NC * _NS
_TOK = _B * _HW
_PER_W = _TOK // _NW
_CH = 128
_NCH = _PER_W // _CH


def _prep_body(eT_ref, eTn_ref, e2_ref):
    eT = eT_ref[...]
    eTn_ref[...] = -2.0 * eT
    e2_ref[...] = jnp.sum(eT * eT, axis=1, keepdims=True)


def _prep_call(embT, interpret=False):
    return pl.pallas_call(
        _prep_body,
        out_shape=[jax.ShapeDtypeStruct((_K, _D), jnp.float32),
                   jax.ShapeDtypeStruct((_K, 1), jnp.float32)],
        interpret=interpret,
    )(embT)


def _argmin_body(z_ref, eTn_ref, e2f_ref, idx_ref):
    z = z_ref[0]
    z2 = jnp.sum(z * z, axis=0, keepdims=True)

    sub_io = lax.broadcasted_iota(jnp.int32, (8, 1), 0)

    def step(k, carry):
        vmin8, varg8 = carry
        e = eTn_ref[pl.ds(k * _KC, _KC), :]
        s2 = lax.dot_general(e, z, (((1,), (0,)), ((), ())),
                             preferred_element_type=jnp.float32)
        e2 = e2f_ref[pl.ds(k * _KC, _KC), :]
        base = k * _KC
        for r in range(_KC // 8):
            sc = (z2 + s2[8 * r:8 * r + 8, :]) + e2[8 * r:8 * r + 8, :]
            idv = sub_io + (base + 8 * r)
            cmp = sc < vmin8
            vmin8 = jnp.where(cmp, sc, vmin8)
            varg8 = jnp.where(cmp, idv, varg8)
        return vmin8, varg8

    init = (jnp.full((8, _HW), jnp.inf, jnp.float32),
            jnp.zeros((8, _HW), jnp.int32))
    vmin8, varg8 = lax.fori_loop(0, _NKC, step, init)

    v, g = vmin8, varg8
    for h in (4, 2, 1):
        va, vb = v[:h, :], v[h:2 * h, :]
        ga, gb = g[:h, :], g[h:2 * h, :]
        tb = (vb < va) | ((vb == va) & (gb < ga))
        v = jnp.where(tb, vb, va)
        g = jnp.where(tb, gb, ga)
    idx_ref[0] = g


def _argmin_call(z3, eTn, e2, interpret=False):
    return pl.pallas_call(
        _argmin_body,
        grid=(_B,),
        in_specs=[pl.BlockSpec((1, _D, _HW), lambda b: (b, 0, 0)),
                  pl.BlockSpec((_K, _D), lambda b: (0, 0)),
                  pl.BlockSpec((_K, 1), lambda b: (0, 0))],
        out_specs=pl.BlockSpec((1, 1, _HW), lambda b: (b, 0, 0)),
        out_shape=jax.ShapeDtypeStruct((_B, 1, _HW), jnp.int32),
        interpret=interpret,
    )(z3, eTn, e2)


def _gather_call(embT, idx_flat):
    mesh = plsc.VectorSubcoreMesh(core_axis_name="c", subcore_axis_name="s")

    @functools.partial(
        pl.kernel, mesh=mesh,
        out_type=jax.ShapeDtypeStruct((_TOK, _D), jnp.float32),
        scratch_types=[pltpu.VMEM((_CH,), jnp.int32),
                       pltpu.VMEM((_CH, _D), jnp.float32),
                       pltpu.SemaphoreType.DMA],
    )
    def k(embT_hbm, idx_hbm, out_hbm, idx_v, rows_v, sem):
        wid = lax.axis_index("s") * _NC + lax.axis_index("c")
        base = wid * _PER_W

        def chunk(c, carry):
            off = base + c * _CH
            pltpu.sync_copy(idx_hbm.at[pl.ds(off, _CH)], idx_v)
            pltpu.async_copy(embT_hbm.at[idx_v], rows_v, sem).wait()
            pltpu.sync_copy(rows_v, out_hbm.at[pl.ds(off, _CH)])
            return carry

        lax.fori_loop(0, _NCH, chunk, 0)

    return k(embT, idx_flat)


def _finish_body(zq_ref, z_ref, out_ref, loss_ref):
    b = pl.program_id(0)
    zqT = zq_ref[0].T
    out_ref[0] = zqT
    d = z_ref[0] - zqT
    part = jnp.full((1, 1), jnp.sum(d * d))

    @pl.when(b == 0)
    def _():
        loss_ref[...] = part

    @pl.when(b > 0)
    def _():
        loss_ref[...] = loss_ref[...] + part

    @pl.when(b == _B - 1)
    def _():
        loss_ref[...] = loss_ref[...] * (_BETA / (_B * _D * _HW))


def _finish_call(zq_tok, z3, interpret=False):
    return pl.pallas_call(
        _finish_body,
        grid=(_B,),
        in_specs=[pl.BlockSpec((1, _HW, _D), lambda b: (b, 0, 0)),
                  pl.BlockSpec((1, _D, _HW), lambda b: (b, 0, 0))],
        out_specs=[pl.BlockSpec((1, _D, _HW), lambda b: (b, 0, 0)),
                   pl.BlockSpec((1, 1), lambda b: (0, 0))],
        out_shape=[jax.ShapeDtypeStruct((_B, _D, _HW), jnp.float32),
                   jax.ShapeDtypeStruct((1, 1), jnp.float32)],
        interpret=interpret,
    )(zq_tok, z3)


def kernel(z_e, embed):
    B, D, H, W = z_e.shape
    z3 = z_e.reshape(B, D, H * W)
    embT = embed.T
    eTn, e2 = _prep_call(embT)
    idx3 = _argmin_call(z3, eTn, e2)
    zq_tok = _gather_call(embT, idx3.reshape(_TOK)).reshape(_B, _HW, _D)
    zq, loss = _finish_call(zq_tok, z3)
    return (zq.reshape(B, D, H, W), idx3.reshape(B, H, W), loss[0, 0])


# speedup vs baseline: 1.9889x; 1.9889x over previous
import jax
import jax.numpy as jnp
from jax import lax
from jax.experimental import pallas as pl
from jax.experimental.pallas import tpu as pltpu

_B = 16
_D = 256
_HW = 1024
_K = 2048
_KC = 256
_NKC = _K // _KC
_BETA = 0.25


def _fused_body(eT_ref, z_ref, zq_ref, idx_ref, loss_ref, e2_ref, oh_ref):
    b = pl.program_id(0)

    @pl.when(b == 0)
    def _():
        eT = eT_ref[...]
        e2_ref[...] = jnp.sum(eT * eT, axis=1, keepdims=True)

    z = z_ref[0]
    sub_io = lax.broadcasted_iota(jnp.int32, (8, 1), 0)

    vmin8 = jnp.full((8, _HW), jnp.inf, jnp.float32)
    varg8 = jnp.zeros((8, _HW), jnp.int32)
    for k in range(_NKC):
        e = eT_ref[pl.ds(k * _KC, _KC), :]
        s = lax.dot_general(e, z, (((1,), (0,)), ((), ())),
                            preferred_element_type=jnp.float32)
        e2 = e2_ref[pl.ds(k * _KC, _KC), :]
        base = k * _KC
        for r in range(_KC // 8):
            sc = e2[8 * r:8 * r + 8, :] - 2.0 * s[8 * r:8 * r + 8, :]
            idv = sub_io + (base + 8 * r)
            cmp = sc < vmin8
            vmin8 = jnp.where(cmp, sc, vmin8)
            varg8 = jnp.where(cmp, idv, varg8)

    v, g = vmin8, varg8
    for h in (4, 2, 1):
        va, vb = v[:h, :], v[h:2 * h, :]
        ga, gb = g[:h, :], g[h:2 * h, :]
        tb = (vb < va) | ((vb == va) & (gb < ga))
        v = jnp.where(tb, vb, va)
        g = jnp.where(tb, gb, ga)
    idx_ref[0] = g

    koto = lax.broadcasted_iota(jnp.int32, (_K, _HW), 0)
    oh_ref[...] = jnp.where(koto == g, 1.0, 0.0)
    zq_ref[0] = lax.dot_general(eT_ref[...], oh_ref[...],
                                (((0,), (0,)), ((), ())),
                                preferred_element_type=jnp.float32)

    part = jnp.full((1, 1), jnp.sum(z * z) + jnp.sum(v))

    @pl.when(b == 0)
    def _():
        loss_ref[...] = part

    @pl.when(b > 0)
    def _():
        loss_ref[...] = loss_ref[...] + part

    @pl.when(b == _B - 1)
    def _():
        loss_ref[...] = loss_ref[...] * (_BETA / (_B * _D * _HW))


def _fused_call(embT, z3, interpret=False):
    return pl.pallas_call(
        _fused_body,
        grid=(_B,),
        in_specs=[pl.BlockSpec((_K, _D), lambda b: (0, 0)),
                  pl.BlockSpec((1, _D, _HW), lambda b: (b, 0, 0))],
        out_specs=[pl.BlockSpec((1, _D, _HW), lambda b: (b, 0, 0)),
                   pl.BlockSpec((1, 1, _HW), lambda b: (b, 0, 0)),
                   pl.BlockSpec((1, 1), lambda b: (0, 0))],
        out_shape=[jax.ShapeDtypeStruct((_B, _D, _HW), jnp.float32),
                   jax.ShapeDtypeStruct((_B, 1, _HW), jnp.int32),
                   jax.ShapeDtypeStruct((1, 1), jnp.float32)],
        scratch_shapes=[pltpu.VMEM((_K, 1), jnp.float32),
                        pltpu.VMEM((_K, _HW), jnp.float32)],
        interpret=interpret,
    )(embT, z3)


def kernel(z_e, embed):
    B, D, H, W = z_e.shape
    z3 = z_e.reshape(B, D, H * W)
    embT = embed.T
    zq, idx3, loss = _fused_call(embT, z3)
    return (zq.reshape(B, D, H, W), idx3.reshape(B, H, W), loss[0, 0])


# speedup vs baseline: 2.0679x; 1.0397x over previous
import jax
import jax.numpy as jnp
from jax import lax
from jax.experimental import pallas as pl
from jax.experimental.pallas import tpu as pltpu

_B = 16
_D = 256
_HW = 1024
_K = 2048
_KC = 256
_NKC = _K // _KC
_BETA = 0.25


def _fused_body(e_ref, z_ref, zq_ref, idx_ref, loss_ref,
                eT_ref, e2_ref, oh_ref):
    b = pl.program_id(0)

    @pl.when(b == 0)
    def _():
        eT = e_ref[...].T
        eT_ref[...] = eT
        e2_ref[...] = jnp.sum(eT * eT, axis=1, keepdims=True)

    z = z_ref[0]
    sub_io = lax.broadcasted_iota(jnp.int32, (8, 1), 0)

    vmin8 = jnp.full((8, _HW), jnp.inf, jnp.float32)
    varg8 = jnp.zeros((8, _HW), jnp.int32)
    for k in range(_NKC):
        e = eT_ref[pl.ds(k * _KC, _KC), :]
        s = lax.dot_general(e, z, (((1,), (0,)), ((), ())),
                            preferred_element_type=jnp.float32)
        e2 = e2_ref[pl.ds(k * _KC, _KC), :]
        base = k * _KC
        for r in range(_KC // 8):
            sc = e2[8 * r:8 * r + 8, :] - 2.0 * s[8 * r:8 * r + 8, :]
            idv = sub_io + (base + 8 * r)
            cmp = sc < vmin8
            vmin8 = jnp.where(cmp, sc, vmin8)
            varg8 = jnp.where(cmp, idv, varg8)

    v, g = vmin8, varg8
    for h in (4, 2, 1):
        va, vb = v[:h, :], v[h:2 * h, :]
        ga, gb = g[:h, :], g[h:2 * h, :]
        tb = (vb < va) | ((vb == va) & (gb < ga))
        v = jnp.where(tb, vb, va)
        g = jnp.where(tb, gb, ga)
    idx_ref[0] = g

    koto = lax.broadcasted_iota(jnp.int32, (_K, _HW), 0)
    oh_ref[...] = jnp.where(koto == g, 1.0, 0.0)
    zq_ref[0] = lax.dot_general(e_ref[...], oh_ref[...],
                                (((1,), (0,)), ((), ())),
                                preferred_element_type=jnp.float32)

    part = jnp.full((1, 1), jnp.sum(z * z) + jnp.sum(v))

    @pl.when(b == 0)
    def _():
        loss_ref[...] = part

    @pl.when(b > 0)
    def _():
        loss_ref[...] = loss_ref[...] + part

    @pl.when(b == _B - 1)
    def _():
        loss_ref[...] = loss_ref[...] * (_BETA / (_B * _D * _HW))


def _fused_call(embed, z3, interpret=False):
    return pl.pallas_call(
        _fused_body,
        grid=(_B,),
        in_specs=[pl.BlockSpec((_D, _K), lambda b: (0, 0)),
                  pl.BlockSpec((1, _D, _HW), lambda b: (b, 0, 0))],
        out_specs=[pl.BlockSpec((1, _D, _HW), lambda b: (b, 0, 0)),
                   pl.BlockSpec((1, 1, _HW), lambda b: (b, 0, 0)),
                   pl.BlockSpec((1, 1), lambda b: (0, 0))],
        out_shape=[jax.ShapeDtypeStruct((_B, _D, _HW), jnp.float32),
                   jax.ShapeDtypeStruct((_B, 1, _HW), jnp.int32),
                   jax.ShapeDtypeStruct((1, 1), jnp.float32)],
        scratch_shapes=[pltpu.VMEM((_K, _D), jnp.float32),
                        pltpu.VMEM((_K, 1), jnp.float32),
                        pltpu.VMEM((_K, _HW), jnp.float32)],
        interpret=interpret,
    )(embed, z3)


def kernel(z_e, embed):
    B, D, H, W = z_e.shape
    z3 = z_e.reshape(B, D, H * W)
    zq, idx3, loss = _fused_call(embed, z3)
    return (zq.reshape(B, D, H, W), idx3.reshape(B, H, W), loss[0, 0])


# speedup vs baseline: 2.1526x; 1.0409x over previous
import jax
import jax.numpy as jnp
from jax import lax
from jax.experimental import pallas as pl
from jax.experimental.pallas import tpu as pltpu

_B = 16
_D = 256
_HW = 1024
_K = 2048
_KC = 256
_NKC = _K // _KC
_BETA = 0.25


def _fused_body(e_ref, z_ref, zq_ref, idx_ref, loss_ref,
                en_ref, e2_ref, oh_ref):
    b = pl.program_id(0)

    @pl.when(b == 0)
    def _():
        eT = e_ref[...].T
        en_ref[...] = -2.0 * eT
        e2_ref[...] = jnp.sum(eT * eT, axis=1, keepdims=True)

    z = z_ref[0]
    sub_io = lax.broadcasted_iota(jnp.int32, (8, 1), 0)

    vmin8 = jnp.full((8, _HW), jnp.inf, jnp.float32)
    varg8 = jnp.zeros((8, _HW), jnp.int32)
    for k in range(_NKC):
        e = en_ref[pl.ds(k * _KC, _KC), :]
        s = lax.dot_general(e, z, (((1,), (0,)), ((), ())),
                            preferred_element_type=jnp.float32)
        e2 = e2_ref[pl.ds(k * _KC, _KC), :]
        base = k * _KC
        for r in range(_KC // 8):
            sc = e2[8 * r:8 * r + 8, :] + s[8 * r:8 * r + 8, :]
            idv = sub_io + (base + 8 * r)
            cmp = sc < vmin8
            vmin8 = jnp.where(cmp, sc, vmin8)
            varg8 = jnp.where(cmp, idv, varg8)

    v, g = vmin8, varg8
    for h in (4, 2, 1):
        va, vb = v[:h, :], v[h:2 * h, :]
        ga, gb = g[:h, :], g[h:2 * h, :]
        tb = (vb < va) | ((vb == va) & (gb < ga))
        v = jnp.where(tb, vb, va)
        g = jnp.where(tb, gb, ga)
    idx_ref[0] = g

    koto = lax.broadcasted_iota(jnp.int32, (_KC, _HW), 0)
    for k in range(_NKC):
        oh_ref[pl.ds(k * _KC, _KC), :] = jnp.where(
            koto == g - k * _KC, 1.0, 0.0)
    zq_ref[0] = lax.dot_general(e_ref[...], oh_ref[...],
                                (((1,), (0,)), ((), ())),
                                preferred_element_type=jnp.float32)

    part = jnp.full((1, 1), jnp.sum(z * z) + jnp.sum(v))

    @pl.when(b == 0)
    def _():
        loss_ref[...] = part

    @pl.when(b > 0)
    def _():
        loss_ref[...] = loss_ref[...] + part

    @pl.when(b == _B - 1)
    def _():
        loss_ref[...] = loss_ref[...] * (_BETA / (_B * _D * _HW))


def _fused_call(embed, z3, interpret=False):
    return pl.pallas_call(
        _fused_body,
        grid=(_B,),
        in_specs=[pl.BlockSpec((_D, _K), lambda b: (0, 0)),
                  pl.BlockSpec((1, _D, _HW), lambda b: (b, 0, 0))],
        out_specs=[pl.BlockSpec((1, _D, _HW), lambda b: (b, 0, 0)),
                   pl.BlockSpec((1, 1, _HW), lambda b: (b, 0, 0)),
                   pl.BlockSpec((1, 1), lambda b: (0, 0))],
        out_shape=[jax.ShapeDtypeStruct((_B, _D, _HW), jnp.float32),
                   jax.ShapeDtypeStruct((_B, 1, _HW), jnp.int32),
                   jax.ShapeDtypeStruct((1, 1), jnp.float32)],
        scratch_shapes=[pltpu.VMEM((_K, _D), jnp.float32),
                        pltpu.VMEM((_K, 1), jnp.float32),
                        pltpu.VMEM((_K, _HW), jnp.float32)],
        interpret=interpret,
    )(embed, z3)


def kernel(z_e, embed):
    B, D, H, W = z_e.shape
    z3 = z_e.reshape(B, D, H * W)
    zq, idx3, loss = _fused_call(embed, z3)
    return (zq.reshape(B, D, H, W), idx3.reshape(B, H, W), loss[0, 0])
